# Initial kernel scaffold; baseline (speedup 1.0000x reference)
import jax
import jax.numpy as jnp
from jax.experimental import pallas as pl


def kernel(embed_drug, embed_disease, W1_t, b1_t, W1_r, b1_r, W2_t, b2_t, W2_r, b2_r, edge_index, neg_dst):
    raise NotImplementedError("write your pallas kernel here")


# speedup vs baseline: 1.2966x; 1.2966x over previous
import functools

import jax
import jax.numpy as jnp
from jax import lax
from jax.experimental import pallas as pl
from jax.experimental.pallas import tpu as pltpu
from jax.experimental.pallas import tpu_sc as plsc

N = 10000
E = 320000
CH = 80
ROWS = E // CH
NSUB = 16
NCORE = 2
RPW_SC = ROWS // NSUB
RPW_ALL = ROWS // (NSUB * NCORE)
NPW = N // NSUB
CW = 128
_WRB = 640
RP = 25
P_ES = RPW_ALL // RP
P_CS = RPW_SC // RP


def _mesh():
    return plsc.VectorSubcoreMesh(core_axis_name="c", subcore_axis_name="s")


def _zero_rows(buf, rows, width):
    zv = jnp.zeros((16,), jnp.float32)

    def body(i, carry):
        for k in range(width // 16):
            buf[i, pl.ds(k * 16, 16)] = zv
        return carry

    lax.fori_loop(0, rows, body, 0)


def _deg_body(src_hbm, dst_hbm, out_hbm, idx_v, ones_v, zb_v, acc):
    cid = lax.axis_index("c")
    sid = lax.axis_index("s")
    ov = jnp.ones((16,), jnp.float32)

    def fill(i, carry):
        for kk in range(CW // 16):
            ones_v[i, pl.ds(kk * 16, 16)] = ov
        return carry

    lax.fori_loop(0, CH, fill, 0)
    _zero_rows(zb_v, CH, CW)
    for k in range(_WRB // CH):
        pltpu.sync_copy(zb_v, acc.at[pl.ds(sid * 624 + k * CH, CH)])
    plsc.subcore_barrier()

    def ppass(p, pcarry):
        @pl.when(cid == 0)
        def _():
            pltpu.sync_copy(src_hbm.at[sid, p], idx_v)

        @pl.when(cid == 1)
        def _():
            pltpu.sync_copy(dst_hbm.at[sid, p], idx_v)

        def chunk(j, carry):
            pltpu.sync_copy(ones_v, acc.at[idx_v.at[j]], add=True)
            return carry

        lax.fori_loop(0, RP, chunk, 0)
        return pcarry

    lax.fori_loop(0, P_CS, ppass, 0)
    plsc.subcore_barrier()
    pltpu.sync_copy(acc.at[pl.ds(sid * 624, _WRB)],
                    out_hbm.at[pl.ds(sid * 624, _WRB), cid])


_deg_kernel = pl.kernel(
    _deg_body,
    out_type=jax.ShapeDtypeStruct((N, NCORE, CW), jnp.float32),
    mesh=_mesh(),
    scratch_types=[
        pltpu.VMEM((RP, CH), jnp.int32),
        pltpu.VMEM((CH, CW), jnp.float32),
        pltpu.VMEM((CH, CW), jnp.float32),
        pltpu.VMEM_SHARED((N, CW), jnp.float32),
    ],
)


def _zero_acc(zb_v, acc, sid, dh):
    _zero_rows(zb_v, CH, dh)
    for k in range(_WRB // CH):
        pltpu.sync_copy(zb_v, acc.at[pl.ds(sid * 624 + k * CH, CH)])


def _agg_main_loop(x_hbm, gi_v, si_v, rb0, rb1, acc, sem0, sem1, rpw):
    def mbody(i, carry):
        j = i * 2
        c0 = pltpu.async_copy(x_hbm.at[gi_v.at[j]], rb0, sem0)
        c1 = pltpu.async_copy(x_hbm.at[gi_v.at[j + 1]], rb1, sem1)
        c0.wait()
        pltpu.sync_copy(rb0, acc.at[si_v.at[j]], add=True)
        c1.wait()
        pltpu.sync_copy(rb1, acc.at[si_v.at[j + 1]], add=True)
        return carry

    lax.fori_loop(0, rpw // 2, mbody, 0)
    if rpw % 2:
        j = rpw - 1
        pltpu.async_copy(x_hbm.at[gi_v.at[j]], rb0, sem0).wait()
        pltpu.sync_copy(rb0, acc.at[si_v.at[j]], add=True)


def _agg_es_body(x_hbm, g_hbm, s_hbm, out_hbm, gi_v, si_v, rb0, rb1,
                 acc, sem0, sem1):
    cid = lax.axis_index("c")
    sid = lax.axis_index("s")
    wid = sid * NCORE + cid
    _zero_acc(rb0, acc, sid, 128)
    plsc.subcore_barrier()

    def pbody(p, carry):
        pltpu.sync_copy(g_hbm.at[wid, p], gi_v)
        pltpu.sync_copy(s_hbm.at[wid, p], si_v)
        _agg_main_loop(x_hbm, gi_v, si_v, rb0, rb1, acc, sem0, sem1, RP)
        return carry

    lax.fori_loop(0, P_ES, pbody, 0)
    plsc.subcore_barrier()
    pltpu.sync_copy(acc.at[pl.ds(sid * 624, _WRB)],
                    out_hbm.at[cid, pl.ds(sid * 624, _WRB)])


_agg_es = pl.kernel(
    _agg_es_body,
    out_type=jax.ShapeDtypeStruct((NCORE, N, 128), jnp.float32),
    mesh=_mesh(),
    scratch_types=[
        pltpu.VMEM((RP, CH), jnp.int32),
        pltpu.VMEM((RP, CH), jnp.int32),
        pltpu.VMEM((CH, 128), jnp.float32),
        pltpu.VMEM((CH, 128), jnp.float32),
        pltpu.VMEM_SHARED((N, 128), jnp.float32),
        pltpu.SemaphoreType.DMA,
        pltpu.SemaphoreType.DMA,
    ],
)


def _agg_cs_body(x_hbm, g_hbm, s_hbm, out_hbm, gi_v, si_v, rb0, rb1,
                 acc, sem0, sem1):
    cid = lax.axis_index("c")
    sid = lax.axis_index("s")
    _zero_acc(rb0, acc, sid, 128)
    plsc.subcore_barrier()

    def pbody(p, carry):
        pltpu.sync_copy(g_hbm.at[sid, p], gi_v)
        pltpu.sync_copy(s_hbm.at[sid, p], si_v)

        def tbody(j, tcarry):
            for k in range(CH // 16):
                v = gi_v[j, pl.ds(k * 16, 16)]
                gi_v[j, pl.ds(k * 16, 16)] = v * 2 + cid
            return tcarry

        lax.fori_loop(0, RP, tbody, 0)
        _agg_main_loop(x_hbm, gi_v, si_v, rb0, rb1, acc, sem0, sem1, RP)
        return carry

    lax.fori_loop(0, P_CS, pbody, 0)
    plsc.subcore_barrier()
    pltpu.sync_copy(acc.at[pl.ds(sid * 624, _WRB)],
                    out_hbm.at[pl.ds(sid * 624, _WRB), cid])


_agg_cs = pl.kernel(
    _agg_cs_body,
    out_type=jax.ShapeDtypeStruct((N, NCORE, 128), jnp.float32),
    mesh=_mesh(),
    scratch_types=[
        pltpu.VMEM((RP, CH), jnp.int32),
        pltpu.VMEM((RP, CH), jnp.int32),
        pltpu.VMEM((CH, 128), jnp.float32),
        pltpu.VMEM((CH, 128), jnp.float32),
        pltpu.VMEM_SHARED((N, 128), jnp.float32),
        pltpu.SemaphoreType.DMA,
        pltpu.SemaphoreType.DMA,
    ],
)


def _pred_body(hd_hbm, hs_hbm, src_hbm, dst_hbm, neg_hbm, pos_hbm, ngo_hbm,
               si_v, di_v, ni_v, ab, bb, cb, pb, nb, sem0, sem1, sem2):
    cid = lax.axis_index("c")
    sid = lax.axis_index("s")
    wid = sid * NCORE + cid

    def ppass(p, pcarry):
        pltpu.sync_copy(src_hbm.at[wid, p], si_v)
        pltpu.sync_copy(dst_hbm.at[wid, p], di_v)
        pltpu.sync_copy(neg_hbm.at[wid, p], ni_v)
        _pred_chunks(hd_hbm, hs_hbm, si_v, di_v, ni_v, ab, bb, cb, pb, nb,
                     sem0, sem1, sem2)
        pltpu.sync_copy(pb, pos_hbm.at[wid, p])
        pltpu.sync_copy(nb, ngo_hbm.at[wid, p])
        return pcarry

    lax.fori_loop(0, P_ES, ppass, 0)


def _pred_chunks(hd_hbm, hs_hbm, si_v, di_v, ni_v, ab, bb, cb, pb, nb,
                 sem0, sem1, sem2):
    def chunk(j, carry):
        ca = pltpu.async_copy(hd_hbm.at[si_v.at[j]], ab, sem0)
        cb_ = pltpu.async_copy(hs_hbm.at[di_v.at[j]], bb, sem1)
        cc = pltpu.async_copy(hs_hbm.at[ni_v.at[j]], cb, sem2)
        ca.wait()
        cb_.wait()
        cc.wait()

        lane = lax.iota(jnp.int32, 16)

        def grp(g, gcarry):
            ridx = g * 16 + lane
            accp = jnp.zeros((16,), jnp.float32)
            accn = jnp.zeros((16,), jnp.float32)
            for k in range(256):
                ck = jnp.full((16,), k, jnp.int32)
                ga = plsc.load_gather(ab, [ridx, ck])
                accp = accp + ga * plsc.load_gather(bb, [ridx, ck])
                accn = accn + ga * plsc.load_gather(cb, [ridx, ck])
            pb[j, pl.ds(g * 16, 16)] = accp
            nb[j, pl.ds(g * 16, 16)] = accn
            return gcarry

        lax.fori_loop(0, CH // 16, grp, 0)
        return carry

    lax.fori_loop(0, RP, chunk, 0)


_pred_kernel = pl.kernel(
    _pred_body,
    out_type=(jax.ShapeDtypeStruct((NSUB * NCORE, P_ES, RP, CH), jnp.float32),
              jax.ShapeDtypeStruct((NSUB * NCORE, P_ES, RP, CH), jnp.float32)),
    mesh=_mesh(),
    compiler_params=pltpu.CompilerParams(needs_layout_passes=False),
    scratch_types=[
        pltpu.VMEM((RP, CH), jnp.int32),
        pltpu.VMEM((RP, CH), jnp.int32),
        pltpu.VMEM((RP, CH), jnp.int32),
        pltpu.VMEM((CH, 256), jnp.float32),
        pltpu.VMEM((CH, 256), jnp.float32),
        pltpu.VMEM((CH, 256), jnp.float32),
        pltpu.VMEM((RP, CH), jnp.float32),
        pltpu.VMEM((RP, CH), jnp.float32),
        pltpu.SemaphoreType.DMA,
        pltpu.SemaphoreType.DMA,
        pltpu.SemaphoreType.DMA,
    ],
)


_RB = 1000


def _prep_body(e_ref, cnt_ref, o_ref):
    r = lax.rsqrt(jnp.maximum(cnt_ref[:, 0:1], 1.0))
    o_ref[...] = e_ref[...] * r


def _prep(e, cnt):
    return pl.pallas_call(
        _prep_body,
        out_shape=jax.ShapeDtypeStruct((N, 128), jnp.float32),
        grid=(N // _RB,),
        in_specs=[
            pl.BlockSpec((_RB, 128), lambda i: (i, 0)),
            pl.BlockSpec((_RB, CW), lambda i: (i, 0)),
        ],
        out_specs=pl.BlockSpec((_RB, 128), lambda i: (i, 0)),
    )(e, cnt)


def _mm_body(relu, scale_next, parts, *refs):
    if scale_next:
        z_ref, cnt_ref, w_ref, b_ref, cnt2_ref, o_ref = refs
    else:
        z_ref, cnt_ref, w_ref, b_ref, o_ref = refs
    t = lax.rsqrt(jnp.maximum(cnt_ref[:, 0:1], 1.0))
    z = (z_ref[0] + z_ref[1]) if parts else z_ref[...]
    a = jnp.dot(z * t, w_ref[...], preferred_element_type=jnp.float32,
                precision=lax.Precision.HIGHEST)
    a = a + b_ref[...]
    if relu:
        a = jnp.maximum(a, 0.0)
    if scale_next:
        a = a * lax.rsqrt(jnp.maximum(cnt2_ref[:, 0:1], 1.0))
    o_ref[...] = a


def _mm(z, cnt, w, b, relu, cnt2=None):
    parts = z.ndim == 3
    d_in = z.shape[-1]
    if parts:
        z_spec = pl.BlockSpec((NCORE, _RB, d_in), lambda i: (0, i, 0))
    else:
        z_spec = pl.BlockSpec((_RB, d_in), lambda i: (i, 0))
    specs = [
        z_spec,
        pl.BlockSpec((_RB, CW), lambda i: (i, 0)),
        pl.BlockSpec((d_in, 256), lambda i: (0, 0)),
        pl.BlockSpec((1, 256), lambda i: (0, 0)),
    ]
    args = [z, cnt, w, b.reshape(1, 256)]
    if cnt2 is not None:
        specs.append(pl.BlockSpec((_RB, CW), lambda i: (i, 0)))
        args.append(cnt2)
    return pl.pallas_call(
        functools.partial(_mm_body, relu, cnt2 is not None, parts),
        out_shape=jax.ShapeDtypeStruct((N, 256), jnp.float32),
        grid=(N // _RB,),
        in_specs=specs,
        out_specs=pl.BlockSpec((_RB, 256), lambda i: (i, 0)),
    )(*args)


def kernel(embed_drug, embed_disease, W1_t, b1_t, W1_r, b1_r, W2_t, b2_t,
           W2_r, b2_r, edge_index, neg_dst):
    src2 = edge_index[0].reshape(NSUB, P_CS, RP, CH)
    dst2 = edge_index[1].reshape(NSUB, P_CS, RP, CH)
    srcp = edge_index[0].reshape(NSUB * NCORE, P_ES, RP, CH)
    dstp = edge_index[1].reshape(NSUB * NCORE, P_ES, RP, CH)
    negp = neg_dst.reshape(NSUB * NCORE, P_ES, RP, CH)

    deg = _deg_kernel(src2, dst2)
    cnt_s = deg[:, 0, :]
    cnt_d = deg[:, 1, :]

    xs_drug = _prep(embed_drug, cnt_s)
    xs_dis = _prep(embed_disease, cnt_d)

    z1t = _agg_es(xs_drug, srcp, dstp)
    z1r = _agg_es(xs_dis, dstp, srcp)

    h_dis_s = _mm(z1t, cnt_d, W1_t, b1_t, relu=True, cnt2=cnt_d)
    h_drug_s = _mm(z1r, cnt_s, W1_r, b1_r, relu=True, cnt2=cnt_s)

    z2t = _agg_cs(h_drug_s.reshape(2 * N, 128), src2, dst2).reshape(N, 256)
    z2r = _agg_cs(h_dis_s.reshape(2 * N, 128), dst2, src2).reshape(N, 256)

    h_dis2 = _mm(z2t, cnt_d, W2_t, b2_t, relu=False)
    h_drug2 = _mm(z2r, cnt_s, W2_r, b2_r, relu=False)

    pos, neg = _pred_kernel(h_drug2, h_dis2, srcp, dstp, negp)
    return pos.reshape(E, 1), neg.reshape(E, 1)


# speedup vs baseline: 3.5005x; 2.6997x over previous
import functools

import jax
import jax.numpy as jnp
from jax import lax
from jax.experimental import pallas as pl
from jax.experimental.pallas import tpu as pltpu
from jax.experimental.pallas import tpu_sc as plsc

N = 10000
E = 320000
CH = 80
ROWS = E // CH
NSUB = 16
NCORE = 2
RPW_SC = ROWS // NSUB
RPW_ALL = ROWS // (NSUB * NCORE)
NPW = N // NSUB
CW = 128
_WRB = 640
RP = 25
P_ES = RPW_ALL // RP
P_CS = RPW_SC // RP


def _mesh():
    return plsc.VectorSubcoreMesh(core_axis_name="c", subcore_axis_name="s")


def _zero_rows(buf, rows, width):
    zv = jnp.zeros((16,), jnp.float32)

    def body(i, carry):
        for k in range(width // 16):
            buf[i, pl.ds(k * 16, 16)] = zv
        return carry

    lax.fori_loop(0, rows, body, 0)


def _deg_body(src_hbm, dst_hbm, out_hbm, idx_v, ones_v, zb_v, acc):
    cid = lax.axis_index("c")
    sid = lax.axis_index("s")
    ov = jnp.ones((16,), jnp.float32)

    def fill(i, carry):
        for kk in range(CW // 16):
            ones_v[i, pl.ds(kk * 16, 16)] = ov
        return carry

    lax.fori_loop(0, CH, fill, 0)
    _zero_rows(zb_v, CH, CW)
    for k in range(_WRB // CH):
        pltpu.sync_copy(zb_v, acc.at[pl.ds(sid * 624 + k * CH, CH)])
    plsc.subcore_barrier()

    def ppass(p, pcarry):
        @pl.when(cid == 0)
        def _():
            pltpu.sync_copy(src_hbm.at[sid, p], idx_v)

        @pl.when(cid == 1)
        def _():
            pltpu.sync_copy(dst_hbm.at[sid, p], idx_v)

        def chunk(j, carry):
            pltpu.sync_copy(ones_v, acc.at[idx_v.at[j]], add=True)
            return carry

        lax.fori_loop(0, RP, chunk, 0)
        return pcarry

    lax.fori_loop(0, P_CS, ppass, 0)
    plsc.subcore_barrier()
    pltpu.sync_copy(acc.at[pl.ds(sid * 624, _WRB)],
                    out_hbm.at[pl.ds(sid * 624, _WRB), cid])


_deg_kernel = pl.kernel(
    _deg_body,
    out_type=jax.ShapeDtypeStruct((N, NCORE, CW), jnp.float32),
    mesh=_mesh(),
    scratch_types=[
        pltpu.VMEM((RP, CH), jnp.int32),
        pltpu.VMEM((CH, CW), jnp.float32),
        pltpu.VMEM((CH, CW), jnp.float32),
        pltpu.VMEM_SHARED((N, CW), jnp.float32),
    ],
)


def _zero_acc(zb_v, acc, sid, dh):
    _zero_rows(zb_v, CH, dh)
    for k in range(_WRB // CH):
        pltpu.sync_copy(zb_v, acc.at[pl.ds(sid * 624 + k * CH, CH)])


def _agg_main_loop(x_hbm, gi_v, si_v, rb0, rb1, acc, sem0, sem1, rpw):
    def mbody(i, carry):
        j = i * 2
        c0 = pltpu.async_copy(x_hbm.at[gi_v.at[j]], rb0, sem0)
        c1 = pltpu.async_copy(x_hbm.at[gi_v.at[j + 1]], rb1, sem1)
        c0.wait()
        pltpu.sync_copy(rb0, acc.at[si_v.at[j]], add=True)
        c1.wait()
        pltpu.sync_copy(rb1, acc.at[si_v.at[j + 1]], add=True)
        return carry

    lax.fori_loop(0, rpw // 2, mbody, 0)
    if rpw % 2:
        j = rpw - 1
        pltpu.async_copy(x_hbm.at[gi_v.at[j]], rb0, sem0).wait()
        pltpu.sync_copy(rb0, acc.at[si_v.at[j]], add=True)


def _agg_es_body(x_hbm, g_hbm, s_hbm, out_hbm, gi_v, si_v, rb0, rb1,
                 acc, sem0, sem1):
    cid = lax.axis_index("c")
    sid = lax.axis_index("s")
    wid = sid * NCORE + cid
    _zero_acc(rb0, acc, sid, 128)
    plsc.subcore_barrier()

    def pbody(p, carry):
        pltpu.sync_copy(g_hbm.at[wid, p], gi_v)
        pltpu.sync_copy(s_hbm.at[wid, p], si_v)
        _agg_main_loop(x_hbm, gi_v, si_v, rb0, rb1, acc, sem0, sem1, RP)
        return carry

    lax.fori_loop(0, P_ES, pbody, 0)
    plsc.subcore_barrier()
    pltpu.sync_copy(acc.at[pl.ds(sid * 624, _WRB)],
                    out_hbm.at[cid, pl.ds(sid * 624, _WRB)])


_agg_es = pl.kernel(
    _agg_es_body,
    out_type=jax.ShapeDtypeStruct((NCORE, N, 128), jnp.float32),
    mesh=_mesh(),
    scratch_types=[
        pltpu.VMEM((RP, CH), jnp.int32),
        pltpu.VMEM((RP, CH), jnp.int32),
        pltpu.VMEM((CH, 128), jnp.float32),
        pltpu.VMEM((CH, 128), jnp.float32),
        pltpu.VMEM_SHARED((N, 128), jnp.float32),
        pltpu.SemaphoreType.DMA,
        pltpu.SemaphoreType.DMA,
    ],
)


def _agg_cs_body(x_hbm, g_hbm, s_hbm, out_hbm, gi_v, si_v, rb0, rb1,
                 acc, sem0, sem1):
    cid = lax.axis_index("c")
    sid = lax.axis_index("s")
    _zero_acc(rb0, acc, sid, 128)
    plsc.subcore_barrier()

    def pbody(p, carry):
        pltpu.sync_copy(g_hbm.at[sid, p], gi_v)
        pltpu.sync_copy(s_hbm.at[sid, p], si_v)

        def tbody(j, tcarry):
            for k in range(CH // 16):
                v = gi_v[j, pl.ds(k * 16, 16)]
                gi_v[j, pl.ds(k * 16, 16)] = v * 2 + cid
            return tcarry

        lax.fori_loop(0, RP, tbody, 0)
        _agg_main_loop(x_hbm, gi_v, si_v, rb0, rb1, acc, sem0, sem1, RP)
        return carry

    lax.fori_loop(0, P_CS, pbody, 0)
    plsc.subcore_barrier()
    pltpu.sync_copy(acc.at[pl.ds(sid * 624, _WRB)],
                    out_hbm.at[pl.ds(sid * 624, _WRB), cid])


_agg_cs = pl.kernel(
    _agg_cs_body,
    out_type=jax.ShapeDtypeStruct((N, NCORE, 128), jnp.float32),
    mesh=_mesh(),
    scratch_types=[
        pltpu.VMEM((RP, CH), jnp.int32),
        pltpu.VMEM((RP, CH), jnp.int32),
        pltpu.VMEM((CH, 128), jnp.float32),
        pltpu.VMEM((CH, 128), jnp.float32),
        pltpu.VMEM_SHARED((N, 128), jnp.float32),
        pltpu.SemaphoreType.DMA,
        pltpu.SemaphoreType.DMA,
    ],
)


def _pred_body(hd_hbm, hs_hbm, src_hbm, dst_hbm, neg_hbm, pos_hbm, ngo_hbm,
               si_v, di_v, ni_v, ab, bb, cb, pb, nb, sem0, sem1, sem2):
    cid = lax.axis_index("c")
    sid = lax.axis_index("s")
    wid = sid * NCORE + cid

    def ppass(p, pcarry):
        pltpu.sync_copy(src_hbm.at[wid, p], si_v)
        pltpu.sync_copy(dst_hbm.at[wid, p], di_v)
        pltpu.sync_copy(neg_hbm.at[wid, p], ni_v)
        _pred_chunks(hd_hbm, hs_hbm, si_v, di_v, ni_v, ab, bb, cb, pb, nb,
                     sem0, sem1, sem2)
        pltpu.sync_copy(pb, pos_hbm.at[wid, p])
        pltpu.sync_copy(nb, ngo_hbm.at[wid, p])
        return pcarry

    lax.fori_loop(0, P_ES, ppass, 0)


def _pred_chunks(hd_hbm, hs_hbm, si_v, di_v, ni_v, ab, bb, cb, pb, nb,
                 sem0, sem1, sem2):
    def chunk(j, carry):
        ca = pltpu.async_copy(hd_hbm.at[si_v.at[j]], ab, sem0)
        cb_ = pltpu.async_copy(hs_hbm.at[di_v.at[j]], bb, sem1)
        cc = pltpu.async_copy(hs_hbm.at[ni_v.at[j]], cb, sem2)
        ca.wait()
        cb_.wait()
        cc.wait()

        lane = lax.iota(jnp.int32, 16)

        def grp(g, gcarry):
            pvec = jnp.zeros((16,), jnp.float32)
            nvec = jnp.zeros((16,), jnp.float32)
            for r in range(16):
                rr = g * 16 + r
                accp = jnp.zeros((16,), jnp.float32)
                accn = jnp.zeros((16,), jnp.float32)
                for k in range(16):
                    av = ab[rr, pl.ds(k * 16, 16)]
                    accp = accp + av * bb[rr, pl.ds(k * 16, 16)]
                    accn = accn + av * cb[rr, pl.ds(k * 16, 16)]
                pvec = jnp.where(lane == r, jnp.sum(accp), pvec)
                nvec = jnp.where(lane == r, jnp.sum(accn), nvec)
            pb[j, pl.ds(g * 16, 16)] = pvec
            nb[j, pl.ds(g * 16, 16)] = nvec
            return gcarry

        lax.fori_loop(0, CH // 16, grp, 0)
        return carry

    lax.fori_loop(0, RP, chunk, 0)


_pred_kernel = pl.kernel(
    _pred_body,
    out_type=(jax.ShapeDtypeStruct((NSUB * NCORE, P_ES, RP, CH), jnp.float32),
              jax.ShapeDtypeStruct((NSUB * NCORE, P_ES, RP, CH), jnp.float32)),
    mesh=_mesh(),
    compiler_params=pltpu.CompilerParams(needs_layout_passes=False),
    scratch_types=[
        pltpu.VMEM((RP, CH), jnp.int32),
        pltpu.VMEM((RP, CH), jnp.int32),
        pltpu.VMEM((RP, CH), jnp.int32),
        pltpu.VMEM((CH, 256), jnp.float32),
        pltpu.VMEM((CH, 256), jnp.float32),
        pltpu.VMEM((CH, 256), jnp.float32),
        pltpu.VMEM((RP, CH), jnp.float32),
        pltpu.VMEM((RP, CH), jnp.float32),
        pltpu.SemaphoreType.DMA,
        pltpu.SemaphoreType.DMA,
        pltpu.SemaphoreType.DMA,
    ],
)


_RB = 1000


def _prep_body(e_ref, cnt_ref, o_ref):
    r = lax.rsqrt(jnp.maximum(cnt_ref[:, 0:1], 1.0))
    o_ref[...] = e_ref[...] * r


def _prep(e, cnt):
    return pl.pallas_call(
        _prep_body,
        out_shape=jax.ShapeDtypeStruct((N, 128), jnp.float32),
        grid=(N // _RB,),
        in_specs=[
            pl.BlockSpec((_RB, 128), lambda i: (i, 0)),
            pl.BlockSpec((_RB, CW), lambda i: (i, 0)),
        ],
        out_specs=pl.BlockSpec((_RB, 128), lambda i: (i, 0)),
    )(e, cnt)


def _mm_body(relu, scale_next, parts, *refs):
    if scale_next:
        z_ref, cnt_ref, w_ref, b_ref, cnt2_ref, o_ref = refs
    else:
        z_ref, cnt_ref, w_ref, b_ref, o_ref = refs
    t = lax.rsqrt(jnp.maximum(cnt_ref[:, 0:1], 1.0))
    z = (z_ref[0] + z_ref[1]) if parts else z_ref[...]
    a = jnp.dot(z * t, w_ref[...], preferred_element_type=jnp.float32,
                precision=lax.Precision.HIGHEST)
    a = a + b_ref[...]
    if relu:
        a = jnp.maximum(a, 0.0)
    if scale_next:
        a = a * lax.rsqrt(jnp.maximum(cnt2_ref[:, 0:1], 1.0))
    o_ref[...] = a


def _mm(z, cnt, w, b, relu, cnt2=None):
    parts = z.ndim == 3
    d_in = z.shape[-1]
    if parts:
        z_spec = pl.BlockSpec((NCORE, _RB, d_in), lambda i: (0, i, 0))
    else:
        z_spec = pl.BlockSpec((_RB, d_in), lambda i: (i, 0))
    specs = [
        z_spec,
        pl.BlockSpec((_RB, CW), lambda i: (i, 0)),
        pl.BlockSpec((d_in, 256), lambda i: (0, 0)),
        pl.BlockSpec((1, 256), lambda i: (0, 0)),
    ]
    args = [z, cnt, w, b.reshape(1, 256)]
    if cnt2 is not None:
        specs.append(pl.BlockSpec((_RB, CW), lambda i: (i, 0)))
        args.append(cnt2)
    return pl.pallas_call(
        functools.partial(_mm_body, relu, cnt2 is not None, parts),
        out_shape=jax.ShapeDtypeStruct((N, 256), jnp.float32),
        grid=(N // _RB,),
        in_specs=specs,
        out_specs=pl.BlockSpec((_RB, 256), lambda i: (i, 0)),
    )(*args)


def kernel(embed_drug, embed_disease, W1_t, b1_t, W1_r, b1_r, W2_t, b2_t,
           W2_r, b2_r, edge_index, neg_dst):
    src2 = edge_index[0].reshape(NSUB, P_CS, RP, CH)
    dst2 = edge_index[1].reshape(NSUB, P_CS, RP, CH)
    srcp = edge_index[0].reshape(NSUB * NCORE, P_ES, RP, CH)
    dstp = edge_index[1].reshape(NSUB * NCORE, P_ES, RP, CH)
    negp = neg_dst.reshape(NSUB * NCORE, P_ES, RP, CH)

    deg = _deg_kernel(src2, dst2)
    cnt_s = deg[:, 0, :]
    cnt_d = deg[:, 1, :]

    xs_drug = _prep(embed_drug, cnt_s)
    xs_dis = _prep(embed_disease, cnt_d)

    z1t = _agg_es(xs_drug, srcp, dstp)
    z1r = _agg_es(xs_dis, dstp, srcp)

    h_dis_s = _mm(z1t, cnt_d, W1_t, b1_t, relu=True, cnt2=cnt_d)
    h_drug_s = _mm(z1r, cnt_s, W1_r, b1_r, relu=True, cnt2=cnt_s)

    z2t = _agg_cs(h_drug_s.reshape(2 * N, 128), src2, dst2).reshape(N, 256)
    z2r = _agg_cs(h_dis_s.reshape(2 * N, 128), dst2, src2).reshape(N, 256)

    h_dis2 = _mm(z2t, cnt_d, W2_t, b2_t, relu=False)
    h_drug2 = _mm(z2r, cnt_s, W2_r, b2_r, relu=False)

    pos, neg = _pred_kernel(h_drug2, h_dis2, srcp, dstp, negp)
    return pos.reshape(E, 1), neg.reshape(E, 1)


# speedup vs baseline: 3.5068x; 1.0018x over previous
import functools

import jax
import jax.numpy as jnp
from jax import lax
from jax.experimental import pallas as pl
from jax.experimental.pallas import tpu as pltpu
from jax.experimental.pallas import tpu_sc as plsc

N = 10000
E = 320000
CH = 80
ROWS = E // CH
NSUB = 16
NCORE = 2
RPW_SC = ROWS // NSUB
RPW_ALL = ROWS // (NSUB * NCORE)
NPW = N // NSUB
CW = 128
_WRB = 640
RP = 25
P_ES = RPW_ALL // RP
P_CS = RPW_SC // RP


def _mesh():
    return plsc.VectorSubcoreMesh(core_axis_name="c", subcore_axis_name="s")


def _zero_rows(buf, rows, width):
    zv = jnp.zeros((16,), jnp.float32)

    def body(i, carry):
        for k in range(width // 16):
            buf[i, pl.ds(k * 16, 16)] = zv
        return carry

    lax.fori_loop(0, rows, body, 0)


def _deg_body(src_hbm, dst_hbm, out_hbm, idx_v, ones_v, zb_v, acc):
    cid = lax.axis_index("c")
    sid = lax.axis_index("s")
    ov = jnp.ones((16,), jnp.float32)

    def fill(i, carry):
        for kk in range(CW // 16):
            ones_v[i, pl.ds(kk * 16, 16)] = ov
        return carry

    lax.fori_loop(0, CH, fill, 0)
    _zero_rows(zb_v, CH, CW)
    for k in range(_WRB // CH):
        pltpu.sync_copy(zb_v, acc.at[pl.ds(sid * 624 + k * CH, CH)])
    plsc.subcore_barrier()

    def ppass(p, pcarry):
        @pl.when(cid == 0)
        def _():
            pltpu.sync_copy(src_hbm.at[sid, p], idx_v)

        @pl.when(cid == 1)
        def _():
            pltpu.sync_copy(dst_hbm.at[sid, p], idx_v)

        def chunk(j, carry):
            pltpu.sync_copy(ones_v, acc.at[idx_v.at[j]], add=True)
            return carry

        lax.fori_loop(0, RP, chunk, 0)
        return pcarry

    lax.fori_loop(0, P_CS, ppass, 0)
    plsc.subcore_barrier()
    pltpu.sync_copy(acc.at[pl.ds(sid * 624, _WRB)],
                    out_hbm.at[pl.ds(sid * 624, _WRB), cid])


_deg_kernel = pl.kernel(
    _deg_body,
    out_type=jax.ShapeDtypeStruct((N, NCORE, CW), jnp.float32),
    mesh=_mesh(),
    scratch_types=[
        pltpu.VMEM((RP, CH), jnp.int32),
        pltpu.VMEM((CH, CW), jnp.float32),
        pltpu.VMEM((CH, CW), jnp.float32),
        pltpu.VMEM_SHARED((N, CW), jnp.float32),
    ],
)


def _zero_acc(zb_v, acc, sid, dh):
    _zero_rows(zb_v, CH, dh)
    for k in range(_WRB // CH):
        pltpu.sync_copy(zb_v, acc.at[pl.ds(sid * 624 + k * CH, CH)])


def _agg_main_loop(x_hbm, gi_v, si_v, rb0, rb1, acc, sem0, sem1, rpw):
    def mbody(i, carry):
        j = i * 2
        c0 = pltpu.async_copy(x_hbm.at[gi_v.at[j]], rb0, sem0)
        c1 = pltpu.async_copy(x_hbm.at[gi_v.at[j + 1]], rb1, sem1)
        c0.wait()
        pltpu.sync_copy(rb0, acc.at[si_v.at[j]], add=True)
        c1.wait()
        pltpu.sync_copy(rb1, acc.at[si_v.at[j + 1]], add=True)
        return carry

    lax.fori_loop(0, rpw // 2, mbody, 0)
    if rpw % 2:
        j = rpw - 1
        pltpu.async_copy(x_hbm.at[gi_v.at[j]], rb0, sem0).wait()
        pltpu.sync_copy(rb0, acc.at[si_v.at[j]], add=True)


def _agg_es_body(x_hbm, g_hbm, s_hbm, out_hbm, gi_v, si_v, rb0, rb1,
                 acc, sem0, sem1):
    cid = lax.axis_index("c")
    sid = lax.axis_index("s")
    wid = sid * NCORE + cid
    _zero_acc(rb0, acc, sid, 128)
    plsc.subcore_barrier()

    def pbody(p, carry):
        pltpu.sync_copy(g_hbm.at[wid, p], gi_v)
        pltpu.sync_copy(s_hbm.at[wid, p], si_v)
        _agg_main_loop(x_hbm, gi_v, si_v, rb0, rb1, acc, sem0, sem1, RP)
        return carry

    lax.fori_loop(0, P_ES, pbody, 0)
    plsc.subcore_barrier()
    pltpu.sync_copy(acc.at[pl.ds(sid * 624, _WRB)],
                    out_hbm.at[cid, pl.ds(sid * 624, _WRB)])


_agg_es = pl.kernel(
    _agg_es_body,
    out_type=jax.ShapeDtypeStruct((NCORE, N, 128), jnp.float32),
    mesh=_mesh(),
    scratch_types=[
        pltpu.VMEM((RP, CH), jnp.int32),
        pltpu.VMEM((RP, CH), jnp.int32),
        pltpu.VMEM((CH, 128), jnp.float32),
        pltpu.VMEM((CH, 128), jnp.float32),
        pltpu.VMEM_SHARED((N, 128), jnp.float32),
        pltpu.SemaphoreType.DMA,
        pltpu.SemaphoreType.DMA,
    ],
)


def _agg_cs_body(x_hbm, g_hbm, s_hbm, out_hbm, gi_v, si_v, rb0, rb1,
                 acc, sem0, sem1):
    cid = lax.axis_index("c")
    sid = lax.axis_index("s")
    _zero_acc(rb0, acc, sid, 128)
    plsc.subcore_barrier()

    def pbody(p, carry):
        pltpu.sync_copy(g_hbm.at[sid, p], gi_v)
        pltpu.sync_copy(s_hbm.at[sid, p], si_v)

        def tbody(j, tcarry):
            for k in range(CH // 16):
                v = gi_v[j, pl.ds(k * 16, 16)]
                gi_v[j, pl.ds(k * 16, 16)] = v * 2 + cid
            return tcarry

        lax.fori_loop(0, RP, tbody, 0)
        _agg_main_loop(x_hbm, gi_v, si_v, rb0, rb1, acc, sem0, sem1, RP)
        return carry

    lax.fori_loop(0, P_CS, pbody, 0)
    plsc.subcore_barrier()
    pltpu.sync_copy(acc.at[pl.ds(sid * 624, _WRB)],
                    out_hbm.at[pl.ds(sid * 624, _WRB), cid])


_agg_cs = pl.kernel(
    _agg_cs_body,
    out_type=jax.ShapeDtypeStruct((N, NCORE, 128), jnp.float32),
    mesh=_mesh(),
    scratch_types=[
        pltpu.VMEM((RP, CH), jnp.int32),
        pltpu.VMEM((RP, CH), jnp.int32),
        pltpu.VMEM((CH, 128), jnp.float32),
        pltpu.VMEM((CH, 128), jnp.float32),
        pltpu.VMEM_SHARED((N, 128), jnp.float32),
        pltpu.SemaphoreType.DMA,
        pltpu.SemaphoreType.DMA,
    ],
)


def _pred_body(hd_hbm, hs_hbm, src_hbm, dst_hbm, neg_hbm, pos_hbm, ngo_hbm,
               si_v, di_v, ni_v, ab, bb, cb, pb, nb, sem0, sem1, sem2):
    cid = lax.axis_index("c")
    sid = lax.axis_index("s")
    wid = sid * NCORE + cid

    def ppass(p, pcarry):
        pltpu.sync_copy(src_hbm.at[wid, p], si_v)
        pltpu.sync_copy(dst_hbm.at[wid, p], di_v)
        pltpu.sync_copy(neg_hbm.at[wid, p], ni_v)
        _pred_chunks(hd_hbm, hs_hbm, si_v, di_v, ni_v, ab, bb, cb, pb, nb,
                     sem0, sem1, sem2)
        pltpu.sync_copy(pb, pos_hbm.at[wid, p])
        pltpu.sync_copy(nb, ngo_hbm.at[wid, p])
        return pcarry

    lax.fori_loop(0, P_ES, ppass, 0)


def _pred_chunks(hd_hbm, hs_hbm, si_v, di_v, ni_v, ab, bb, cb, pb, nb,
                 sem0, sem1, sem2):
    def chunk(j, carry):
        ca = pltpu.async_copy(hd_hbm.at[si_v.at[j]], ab, sem0)
        cb_ = pltpu.async_copy(hs_hbm.at[di_v.at[j]], bb, sem1)
        cc = pltpu.async_copy(hs_hbm.at[ni_v.at[j]], cb, sem2)
        ca.wait()
        cb_.wait()
        cc.wait()

        lane = lax.iota(jnp.int32, 16)

        def grp(g, gcarry):
            pvec = jnp.zeros((16,), jnp.float32)
            nvec = jnp.zeros((16,), jnp.float32)
            for r in range(16):
                rr = g * 16 + r
                accp = jnp.zeros((16,), jnp.float32)
                accn = jnp.zeros((16,), jnp.float32)
                for k in range(16):
                    av = ab[rr, pl.ds(k * 16, 16)]
                    accp = accp + av * bb[rr, pl.ds(k * 16, 16)]
                    accn = accn + av * cb[rr, pl.ds(k * 16, 16)]
                pvec = jnp.where(lane == r, jnp.sum(accp), pvec)
                nvec = jnp.where(lane == r, jnp.sum(accn), nvec)
            pb[j, pl.ds(g * 16, 16)] = pvec
            nb[j, pl.ds(g * 16, 16)] = nvec
            return gcarry

        lax.fori_loop(0, CH // 16, grp, 0)
        return carry

    lax.fori_loop(0, RP, chunk, 0)


_pred_kernel = pl.kernel(
    _pred_body,
    out_type=(jax.ShapeDtypeStruct((NSUB * NCORE, P_ES, RP, CH), jnp.float32),
              jax.ShapeDtypeStruct((NSUB * NCORE, P_ES, RP, CH), jnp.float32)),
    mesh=_mesh(),
    compiler_params=pltpu.CompilerParams(needs_layout_passes=False),
    scratch_types=[
        pltpu.VMEM((RP, CH), jnp.int32),
        pltpu.VMEM((RP, CH), jnp.int32),
        pltpu.VMEM((RP, CH), jnp.int32),
        pltpu.VMEM((CH, 256), jnp.float32),
        pltpu.VMEM((CH, 256), jnp.float32),
        pltpu.VMEM((CH, 256), jnp.float32),
        pltpu.VMEM((RP, CH), jnp.float32),
        pltpu.VMEM((RP, CH), jnp.float32),
        pltpu.SemaphoreType.DMA,
        pltpu.SemaphoreType.DMA,
        pltpu.SemaphoreType.DMA,
    ],
)


_RB = 1000


def _prep_body(e_ref, cnt_ref, o_ref):
    r = jnp.power(jnp.maximum(cnt_ref[:, 0:1], 1.0), -0.5)
    o_ref[...] = e_ref[...] * r


def _prep(e, cnt):
    return pl.pallas_call(
        _prep_body,
        out_shape=jax.ShapeDtypeStruct((N, 128), jnp.float32),
        grid=(N // _RB,),
        in_specs=[
            pl.BlockSpec((_RB, 128), lambda i: (i, 0)),
            pl.BlockSpec((_RB, CW), lambda i: (i, 0)),
        ],
        out_specs=pl.BlockSpec((_RB, 128), lambda i: (i, 0)),
    )(e, cnt)


def _mm_body(relu, scale_next, parts, *refs):
    if scale_next:
        z_ref, cnt_ref, w_ref, b_ref, cnt2_ref, o_ref = refs
    else:
        z_ref, cnt_ref, w_ref, b_ref, o_ref = refs
    t = jnp.power(jnp.maximum(cnt_ref[:, 0:1], 1.0), -0.5)
    z = (z_ref[0] + z_ref[1]) if parts else z_ref[...]
    a = jnp.dot(z * t, w_ref[...], preferred_element_type=jnp.float32,
                precision=lax.Precision.HIGHEST)
    a = a + b_ref[...]
    if relu:
        a = jnp.maximum(a, 0.0)
    if scale_next:
        a = a * jnp.power(jnp.maximum(cnt2_ref[:, 0:1], 1.0), -0.5)
    o_ref[...] = a


def _mm(z, cnt, w, b, relu, cnt2=None):
    parts = z.ndim == 3
    d_in = z.shape[-1]
    if parts:
        z_spec = pl.BlockSpec((NCORE, _RB, d_in), lambda i: (0, i, 0))
    else:
        z_spec = pl.BlockSpec((_RB, d_in), lambda i: (i, 0))
    specs = [
        z_spec,
        pl.BlockSpec((_RB, CW), lambda i: (i, 0)),
        pl.BlockSpec((d_in, 256), lambda i: (0, 0)),
        pl.BlockSpec((1, 256), lambda i: (0, 0)),
    ]
    args = [z, cnt, w, b.reshape(1, 256)]
    if cnt2 is not None:
        specs.append(pl.BlockSpec((_RB, CW), lambda i: (i, 0)))
        args.append(cnt2)
    return pl.pallas_call(
        functools.partial(_mm_body, relu, cnt2 is not None, parts),
        out_shape=jax.ShapeDtypeStruct((N, 256), jnp.float32),
        grid=(N // _RB,),
        in_specs=specs,
        out_specs=pl.BlockSpec((_RB, 256), lambda i: (i, 0)),
    )(*args)


def kernel(embed_drug, embed_disease, W1_t, b1_t, W1_r, b1_r, W2_t, b2_t,
           W2_r, b2_r, edge_index, neg_dst):
    src2 = edge_index[0].reshape(NSUB, P_CS, RP, CH)
    dst2 = edge_index[1].reshape(NSUB, P_CS, RP, CH)
    srcp = edge_index[0].reshape(NSUB * NCORE, P_ES, RP, CH)
    dstp = edge_index[1].reshape(NSUB * NCORE, P_ES, RP, CH)
    negp = neg_dst.reshape(NSUB * NCORE, P_ES, RP, CH)

    deg = _deg_kernel(src2, dst2)
    cnt_s = deg[:, 0, :]
    cnt_d = deg[:, 1, :]

    xs_drug = _prep(embed_drug, cnt_s)
    xs_dis = _prep(embed_disease, cnt_d)

    z1t = _agg_es(xs_drug, srcp, dstp)
    z1r = _agg_es(xs_dis, dstp, srcp)

    h_dis_s = _mm(z1t, cnt_d, W1_t, b1_t, relu=True, cnt2=cnt_d)
    h_drug_s = _mm(z1r, cnt_s, W1_r, b1_r, relu=True, cnt2=cnt_s)

    z2t = _agg_cs(h_drug_s.reshape(2 * N, 128), src2, dst2).reshape(N, 256)
    z2r = _agg_cs(h_dis_s.reshape(2 * N, 128), dst2, src2).reshape(N, 256)

    h_dis2 = _mm(z2t, cnt_d, W2_t, b2_t, relu=False)
    h_drug2 = _mm(z2r, cnt_s, W2_r, b2_r, relu=False)

    pos, neg = _pred_kernel(h_drug2, h_dis2, srcp, dstp, negp)
    return pos.reshape(E, 1), neg.reshape(E, 1)


# speedup vs baseline: 4.2275x; 1.2055x over previous
import functools

import jax
import jax.numpy as jnp
from jax import lax
from jax.experimental import pallas as pl
from jax.experimental.pallas import tpu as pltpu
from jax.experimental.pallas import tpu_sc as plsc

N = 10000
E = 320000
CH = 80
ROWS = E // CH
NSUB = 16
NCORE = 2
RPW_SC = ROWS // NSUB
RPW_ALL = ROWS // (NSUB * NCORE)
NPW = N // NSUB
CW = 128
_WRB = 640
RP = 25
P_ES = RPW_ALL // RP
P_CS = RPW_SC // RP


def _mesh():
    return plsc.VectorSubcoreMesh(core_axis_name="c", subcore_axis_name="s")


def _zero_rows(buf, rows, width):
    zv = jnp.zeros((16,), jnp.float32)

    def body(i, carry):
        for k in range(width // 16):
            buf[i, pl.ds(k * 16, 16)] = zv
        return carry

    lax.fori_loop(0, rows, body, 0)


def _deg_body(src_hbm, dst_hbm, out_hbm, idx_v, ones_v, zb_v, acc):
    cid = lax.axis_index("c")
    sid = lax.axis_index("s")
    ov = jnp.ones((16,), jnp.float32)

    def fill(i, carry):
        for kk in range(CW // 16):
            ones_v[i, pl.ds(kk * 16, 16)] = ov
        return carry

    lax.fori_loop(0, CH, fill, 0)
    _zero_rows(zb_v, CH, CW)
    for k in range(_WRB // CH):
        pltpu.sync_copy(zb_v, acc.at[pl.ds(sid * 624 + k * CH, CH)])
    plsc.subcore_barrier()

    def ppass(p, pcarry):
        @pl.when(cid == 0)
        def _():
            pltpu.sync_copy(src_hbm.at[sid, p], idx_v)

        @pl.when(cid == 1)
        def _():
            pltpu.sync_copy(dst_hbm.at[sid, p], idx_v)

        def chunk(j, carry):
            pltpu.sync_copy(ones_v, acc.at[idx_v.at[j]], add=True)
            return carry

        lax.fori_loop(0, RP, chunk, 0)
        return pcarry

    lax.fori_loop(0, P_CS, ppass, 0)
    plsc.subcore_barrier()
    pltpu.sync_copy(acc.at[pl.ds(sid * 624, _WRB)],
                    out_hbm.at[pl.ds(sid * 624, _WRB), cid])


_deg_kernel = pl.kernel(
    _deg_body,
    out_type=jax.ShapeDtypeStruct((N, NCORE, CW), jnp.float32),
    mesh=_mesh(),
    scratch_types=[
        pltpu.VMEM((RP, CH), jnp.int32),
        pltpu.VMEM((CH, CW), jnp.float32),
        pltpu.VMEM((CH, CW), jnp.float32),
        pltpu.VMEM_SHARED((N, CW), jnp.float32),
    ],
)


def _zero_acc(zb_v, acc, sid, dh):
    _zero_rows(zb_v, CH, dh)
    for k in range(_WRB // CH):
        pltpu.sync_copy(zb_v, acc.at[pl.ds(sid * 624 + k * CH, CH)])


def _agg_main_loop(x_hbm, gi_v, si_v, rb0, rb1, acc, sg0, sg1, ss0, ss1, rpw):
    assert rpw % 2 == 1
    pltpu.async_copy(x_hbm.at[gi_v.at[0]], rb0, sg0)
    pltpu.async_copy(x_hbm.at[gi_v.at[1]], rb1, sg1)

    def mbody(i, carry):
        j = i * 2
        pltpu.make_async_copy(x_hbm.at[gi_v.at[j]], rb0, sg0).wait()
        s0 = pltpu.async_copy(rb0, acc.at[si_v.at[j]], ss0, add=True)
        pltpu.make_async_copy(x_hbm.at[gi_v.at[j + 1]], rb1, sg1).wait()
        s1 = pltpu.async_copy(rb1, acc.at[si_v.at[j + 1]], ss1, add=True)
        s0.wait()
        pltpu.async_copy(x_hbm.at[gi_v.at[j + 2]], rb0, sg0)
        s1.wait()

        @pl.when(j + 3 < rpw)
        def _():
            pltpu.async_copy(x_hbm.at[gi_v.at[j + 3]], rb1, sg1)

        return carry

    lax.fori_loop(0, (rpw - 1) // 2, mbody, 0)
    j = rpw - 1
    pltpu.make_async_copy(x_hbm.at[gi_v.at[j]], rb0, sg0).wait()
    pltpu.async_copy(rb0, acc.at[si_v.at[j]], ss0, add=True).wait()


def _agg_es_body(x_hbm, g_hbm, s_hbm, out_hbm, gi_v, si_v, rb0, rb1,
                 acc, sg0, sg1, ss0, ss1):
    cid = lax.axis_index("c")
    sid = lax.axis_index("s")
    wid = sid * NCORE + cid
    _zero_acc(rb0, acc, sid, 128)
    plsc.subcore_barrier()

    def pbody(p, carry):
        pltpu.sync_copy(g_hbm.at[wid, p], gi_v)
        pltpu.sync_copy(s_hbm.at[wid, p], si_v)
        _agg_main_loop(x_hbm, gi_v, si_v, rb0, rb1, acc, sg0, sg1, ss0, ss1, RP)
        return carry

    lax.fori_loop(0, P_ES, pbody, 0)
    plsc.subcore_barrier()
    pltpu.sync_copy(acc.at[pl.ds(sid * 624, _WRB)],
                    out_hbm.at[cid, pl.ds(sid * 624, _WRB)])


_agg_es = pl.kernel(
    _agg_es_body,
    out_type=jax.ShapeDtypeStruct((NCORE, N, 128), jnp.float32),
    mesh=_mesh(),
    scratch_types=[
        pltpu.VMEM((RP, CH), jnp.int32),
        pltpu.VMEM((RP, CH), jnp.int32),
        pltpu.VMEM((CH, 128), jnp.float32),
        pltpu.VMEM((CH, 128), jnp.float32),
        pltpu.VMEM_SHARED((N, 128), jnp.float32),
        pltpu.SemaphoreType.DMA,
        pltpu.SemaphoreType.DMA,
        pltpu.SemaphoreType.DMA,
        pltpu.SemaphoreType.DMA,
    ],
)


def _agg_cs_body(x_hbm, g_hbm, s_hbm, out_hbm, gi_v, si_v, rb0, rb1,
                 acc, sg0, sg1, ss0, ss1):
    cid = lax.axis_index("c")
    sid = lax.axis_index("s")
    _zero_acc(rb0, acc, sid, 128)
    plsc.subcore_barrier()

    def pbody(p, carry):
        pltpu.sync_copy(g_hbm.at[sid, p], gi_v)
        pltpu.sync_copy(s_hbm.at[sid, p], si_v)

        def tbody(j, tcarry):
            for k in range(CH // 16):
                v = gi_v[j, pl.ds(k * 16, 16)]
                gi_v[j, pl.ds(k * 16, 16)] = v * 2 + cid
            return tcarry

        lax.fori_loop(0, RP, tbody, 0)
        _agg_main_loop(x_hbm, gi_v, si_v, rb0, rb1, acc, sg0, sg1, ss0, ss1, RP)
        return carry

    lax.fori_loop(0, P_CS, pbody, 0)
    plsc.subcore_barrier()
    pltpu.sync_copy(acc.at[pl.ds(sid * 624, _WRB)],
                    out_hbm.at[pl.ds(sid * 624, _WRB), cid])


_agg_cs = pl.kernel(
    _agg_cs_body,
    out_type=jax.ShapeDtypeStruct((N, NCORE, 128), jnp.float32),
    mesh=_mesh(),
    scratch_types=[
        pltpu.VMEM((RP, CH), jnp.int32),
        pltpu.VMEM((RP, CH), jnp.int32),
        pltpu.VMEM((CH, 128), jnp.float32),
        pltpu.VMEM((CH, 128), jnp.float32),
        pltpu.VMEM_SHARED((N, 128), jnp.float32),
        pltpu.SemaphoreType.DMA,
        pltpu.SemaphoreType.DMA,
        pltpu.SemaphoreType.DMA,
        pltpu.SemaphoreType.DMA,
    ],
)


def _pred_half(abuf, bbuf, cbuf, pb, nb, j, first):
    lane = lax.iota(jnp.int32, 16)

    def grp(g, gcarry):
        pvec = jnp.zeros((16,), jnp.float32)
        nvec = jnp.zeros((16,), jnp.float32)
        for r in range(16):
            rr = g * 16 + r
            accp = jnp.zeros((16,), jnp.float32)
            accn = jnp.zeros((16,), jnp.float32)
            for k in range(8):
                av = abuf[rr, pl.ds(k * 16, 16)]
                accp = accp + av * bbuf[rr, pl.ds(k * 16, 16)]
                accn = accn + av * cbuf[rr, pl.ds(k * 16, 16)]
            pvec = jnp.where(lane == r, jnp.sum(accp), pvec)
            nvec = jnp.where(lane == r, jnp.sum(accn), nvec)
        sl = pl.ds(g * 16, 16)
        if first:
            pb[j, sl] = pvec
            nb[j, sl] = nvec
        else:
            pb[j, sl] = pb[j, sl] + pvec
            nb[j, sl] = nb[j, sl] + nvec
        return gcarry

    lax.fori_loop(0, CH // 16, grp, 0)


def _pred_body(hd_hbm, hs_hbm, src_hbm, dst_hbm, neg_hbm, pos_hbm, ngo_hbm,
               si_v, di_v, ni_v, se_v, de_v, ne_v, a0, b0, c0, a1, b1, c1,
               pb, nb, sma0, smb0, smc0, sma1, smb1, smc1):
    cid = lax.axis_index("c")
    sid = lax.axis_index("s")
    wid = sid * NCORE + cid

    def fire0(j):
        return (pltpu.async_copy(hd_hbm.at[se_v.at[j]], a0, sma0),
                pltpu.async_copy(hs_hbm.at[de_v.at[j]], b0, smb0),
                pltpu.async_copy(hs_hbm.at[ne_v.at[j]], c0, smc0))

    def fire1(j):
        return (pltpu.async_copy(hd_hbm.at[si_v.at[j]], a1, sma1),
                pltpu.async_copy(hs_hbm.at[di_v.at[j]], b1, smb1),
                pltpu.async_copy(hs_hbm.at[ni_v.at[j]], c1, smc1))

    def ppass(p, pcarry):
        pltpu.sync_copy(src_hbm.at[wid, p], si_v)
        pltpu.sync_copy(dst_hbm.at[wid, p], di_v)
        pltpu.sync_copy(neg_hbm.at[wid, p], ni_v)

        def tbody(j, tcarry):
            for k in range(CH // 16):
                sl = pl.ds(k * 16, 16)
                for raw, ev in ((si_v, se_v), (di_v, de_v), (ni_v, ne_v)):
                    rr = raw.at[j]
                    er = ev.at[j]
                    v2 = rr[sl] * 2
                    er[sl] = v2
                    rr[sl] = v2 + 1
            return tcarry

        lax.fori_loop(0, RP, tbody, 0)
        fire0(0)

        def chunk(j, ccarry):
            fire1(j)
            pltpu.make_async_copy(hd_hbm.at[se_v.at[j]], a0, sma0).wait()
            pltpu.make_async_copy(hs_hbm.at[de_v.at[j]], b0, smb0).wait()
            pltpu.make_async_copy(hs_hbm.at[ne_v.at[j]], c0, smc0).wait()
            _pred_half(a0, b0, c0, pb, nb, j, first=True)

            @pl.when(j + 1 < RP)
            def _():
                fire0(j + 1)

            pltpu.make_async_copy(hd_hbm.at[si_v.at[j]], a1, sma1).wait()
            pltpu.make_async_copy(hs_hbm.at[di_v.at[j]], b1, smb1).wait()
            pltpu.make_async_copy(hs_hbm.at[ni_v.at[j]], c1, smc1).wait()
            _pred_half(a1, b1, c1, pb, nb, j, first=False)
            return ccarry

        lax.fori_loop(0, RP, chunk, 0)
        pltpu.sync_copy(pb, pos_hbm.at[wid, p])
        pltpu.sync_copy(nb, ngo_hbm.at[wid, p])
        return pcarry

    lax.fori_loop(0, P_ES, ppass, 0)


_pred_kernel = pl.kernel(
    _pred_body,
    out_type=(jax.ShapeDtypeStruct((NSUB * NCORE, P_ES, RP, CH), jnp.float32),
              jax.ShapeDtypeStruct((NSUB * NCORE, P_ES, RP, CH), jnp.float32)),
    mesh=_mesh(),
    compiler_params=pltpu.CompilerParams(needs_layout_passes=False),
    scratch_types=[
        pltpu.VMEM((RP, CH), jnp.int32),
        pltpu.VMEM((RP, CH), jnp.int32),
        pltpu.VMEM((RP, CH), jnp.int32),
        pltpu.VMEM((RP, CH), jnp.int32),
        pltpu.VMEM((RP, CH), jnp.int32),
        pltpu.VMEM((RP, CH), jnp.int32),
        pltpu.VMEM((CH, 128), jnp.float32),
        pltpu.VMEM((CH, 128), jnp.float32),
        pltpu.VMEM((CH, 128), jnp.float32),
        pltpu.VMEM((CH, 128), jnp.float32),
        pltpu.VMEM((CH, 128), jnp.float32),
        pltpu.VMEM((CH, 128), jnp.float32),
        pltpu.VMEM((RP, CH), jnp.float32),
        pltpu.VMEM((RP, CH), jnp.float32),
        pltpu.SemaphoreType.DMA,
        pltpu.SemaphoreType.DMA,
        pltpu.SemaphoreType.DMA,
        pltpu.SemaphoreType.DMA,
        pltpu.SemaphoreType.DMA,
        pltpu.SemaphoreType.DMA,
    ],
)


_RB = 1000


def _prep_body(e_ref, cnt_ref, o_ref):
    r = jnp.power(jnp.maximum(cnt_ref[:, 0:1], 1.0), -0.5)
    o_ref[...] = e_ref[...] * r


def _prep(e, cnt):
    return pl.pallas_call(
        _prep_body,
        out_shape=jax.ShapeDtypeStruct((N, 128), jnp.float32),
        grid=(N // _RB,),
        in_specs=[
            pl.BlockSpec((_RB, 128), lambda i: (i, 0)),
            pl.BlockSpec((_RB, CW), lambda i: (i, 0)),
        ],
        out_specs=pl.BlockSpec((_RB, 128), lambda i: (i, 0)),
    )(e, cnt)


def _mm_body(relu, scale_next, parts, *refs):
    if scale_next:
        z_ref, cnt_ref, w_ref, b_ref, cnt2_ref, o_ref = refs
    else:
        z_ref, cnt_ref, w_ref, b_ref, o_ref = refs
    t = jnp.power(jnp.maximum(cnt_ref[:, 0:1], 1.0), -0.5)
    z = (z_ref[0] + z_ref[1]) if parts else z_ref[...]
    a = jnp.dot(z * t, w_ref[...], preferred_element_type=jnp.float32,
                precision=lax.Precision.HIGHEST)
    a = a + b_ref[...]
    if relu:
        a = jnp.maximum(a, 0.0)
    if scale_next:
        a = a * jnp.power(jnp.maximum(cnt2_ref[:, 0:1], 1.0), -0.5)
    o_ref[...] = a


def _mm(z, cnt, w, b, relu, cnt2=None):
    parts = z.ndim == 3
    d_in = z.shape[-1]
    if parts:
        z_spec = pl.BlockSpec((NCORE, _RB, d_in), lambda i: (0, i, 0))
    else:
        z_spec = pl.BlockSpec((_RB, d_in), lambda i: (i, 0))
    specs = [
        z_spec,
        pl.BlockSpec((_RB, CW), lambda i: (i, 0)),
        pl.BlockSpec((d_in, 256), lambda i: (0, 0)),
        pl.BlockSpec((1, 256), lambda i: (0, 0)),
    ]
    args = [z, cnt, w, b.reshape(1, 256)]
    if cnt2 is not None:
        specs.append(pl.BlockSpec((_RB, CW), lambda i: (i, 0)))
        args.append(cnt2)
    return pl.pallas_call(
        functools.partial(_mm_body, relu, cnt2 is not None, parts),
        out_shape=jax.ShapeDtypeStruct((N, 256), jnp.float32),
        grid=(N // _RB,),
        in_specs=specs,
        out_specs=pl.BlockSpec((_RB, 256), lambda i: (i, 0)),
    )(*args)


def kernel(embed_drug, embed_disease, W1_t, b1_t, W1_r, b1_r, W2_t, b2_t,
           W2_r, b2_r, edge_index, neg_dst):
    src2 = edge_index[0].reshape(NSUB, P_CS, RP, CH)
    dst2 = edge_index[1].reshape(NSUB, P_CS, RP, CH)
    srcp = edge_index[0].reshape(NSUB * NCORE, P_ES, RP, CH)
    dstp = edge_index[1].reshape(NSUB * NCORE, P_ES, RP, CH)
    negp = neg_dst.reshape(NSUB * NCORE, P_ES, RP, CH)

    deg = _deg_kernel(src2, dst2)
    cnt_s = deg[:, 0, :]
    cnt_d = deg[:, 1, :]

    xs_drug = _prep(embed_drug, cnt_s)
    xs_dis = _prep(embed_disease, cnt_d)

    z1t = _agg_es(xs_drug, srcp, dstp)
    z1r = _agg_es(xs_dis, dstp, srcp)

    h_dis_s = _mm(z1t, cnt_d, W1_t, b1_t, relu=True, cnt2=cnt_d)
    h_drug_s = _mm(z1r, cnt_s, W1_r, b1_r, relu=True, cnt2=cnt_s)

    z2t = _agg_cs(h_drug_s.reshape(2 * N, 128), src2, dst2).reshape(N, 256)
    z2r = _agg_cs(h_dis_s.reshape(2 * N, 128), dst2, src2).reshape(N, 256)

    h_dis2 = _mm(z2t, cnt_d, W2_t, b2_t, relu=False)
    h_drug2 = _mm(z2r, cnt_s, W2_r, b2_r, relu=False)

    pos, neg = _pred_kernel(h_drug2.reshape(2 * N, 128),
                            h_dis2.reshape(2 * N, 128), srcp, dstp, negp)
    return pos.reshape(E, 1), neg.reshape(E, 1)
